# baseline (speedup 1.0000x reference)
import functools

import jax
import jax.numpy as jnp
from jax import lax
from jax.experimental import pallas as pl
from jax.experimental.pallas import tpu as pltpu
from jax.experimental.pallas import tpu_sc as plsc


E_BLK = 256
S_BLK = 64

_ROT = ((13, 15, 26, 6), (17, 29, 16, 24))
_C240 = 0x1BD11BDA


def _threefry2x32(k0, k1, x0, x1):
    ks = (k0, k1, k0 ^ k1 ^ jnp.int32(_C240))
    for i in range(5):
        for r in _ROT[i % 2]:
            x0 = x0 + x1
            x1 = (x1 << r) + lax.shift_right_logical(x1, 32 - r)
            x1 = x1 ^ x0
        inj0 = ks[(i + 1) % 3]
        if i == 4:
            inj0 = inj0 + jnp.int32(-(2**31))
        x0 = x0 + inj0
        x1 = x1 + ks[(i + 2) % 3] + jnp.int32(i + 1)
    return x0, x1


def _sample_kernel(keys_ref, idx_ref, *, seq_len, emb):
    b = pl.program_id(0)
    k0 = keys_ref[b, 0]
    k1 = keys_ref[b, 1]

    n_schunk = seq_len // S_BLK

    sub = lax.broadcasted_iota(jnp.int32, (S_BLK, E_BLK), 0)
    lane = lax.broadcasted_iota(jnp.int32, (S_BLK, E_BLK), 1)
    sb_bits = S_BLK.bit_length() - 1
    revsub = (S_BLK - 1) - sub

    x0_init = jnp.full((S_BLK, E_BLK), k0, dtype=jnp.int32)

    for ec in range(0, emb, E_BLK):
        f0k = (lane + ec) * seq_len + sub + k1

        def one_chunk(s0, carry, f0k=f0k):
            run_max, run_idx = carry
            o0, o1 = _threefry2x32(k0, k1, x0_init, f0k + s0)
            y = o0 ^ o1
            packed = (y & jnp.int32(-512)) + revsub
            c_pack = jnp.max(packed, axis=0, keepdims=True)
            c_max = c_pack & jnp.int32(-512)
            c_idx = (s0 + (S_BLK - 1)) - (c_pack & (S_BLK - 1))
            upd = c_max > run_max
            return (
                jnp.where(upd, c_max, run_max),
                jnp.where(upd, c_idx, run_idx),
            )

        def s_body(si, carry):
            s0 = si * (2 * S_BLK)
            return one_chunk(s0 + S_BLK, one_chunk(s0, carry))

        init = (
            jnp.full((1, E_BLK), -(2**31), dtype=jnp.int32),
            jnp.zeros((1, E_BLK), dtype=jnp.int32),
        )
        _, s_star = lax.fori_loop(0, n_schunk // 2, s_body, init)
        lane_row = lax.broadcasted_iota(jnp.int32, (1, E_BLK), 1) + ec
        idx_ref[0, :, pl.ds(ec, E_BLK)] = (
            (b * seq_len + s_star) * emb + lane_row
        )


def _make_sc_gather(n_idx, per_w, n_workers, num_cores):
    mesh = plsc.VectorSubcoreMesh(core_axis_name="c", subcore_axis_name="s")

    @functools.partial(
        pl.kernel,
        mesh=mesh,
        out_type=jax.ShapeDtypeStruct((n_idx,), jnp.float32),
        scratch_types=[
            pltpu.VMEM((per_w,), jnp.int32),
            pltpu.VMEM((per_w,), jnp.float32),
            pltpu.SemaphoreType.DMA,
        ],
    )
    def sc_gather(tokens_hbm, idx_hbm, out_hbm, idx_v, vals_v, sem):
        wid = lax.axis_index("s") * num_cores + lax.axis_index("c")
        base = wid * per_w
        pltpu.sync_copy(idx_hbm.at[pl.ds(base, per_w)], idx_v)
        pltpu.async_copy(tokens_hbm.at[idx_v], vals_v, sem).wait()
        pltpu.sync_copy(vals_v, out_hbm.at[pl.ds(base, per_w)])

    return sc_gather


SC_ROWS = 15


def _make_sc_sample(bs, seq_len, emb, k_rows, n_workers, num_cores):
    mesh = plsc.VectorSubcoreMesh(core_axis_name="c", subcore_axis_name="s")
    per_w = (k_rows * emb) // n_workers
    n_groups = per_w // 16

    @functools.partial(
        pl.kernel,
        mesh=mesh,
        out_type=jax.ShapeDtypeStruct((k_rows * emb,), jnp.int32),
        scratch_types=[
            pltpu.VMEM((16,), jnp.int32),
            pltpu.VMEM((16,), jnp.int32),
            pltpu.VMEM((per_w,), jnp.int32),
        ],
    )
    def sc_sample(k0_hbm, k1_hbm, out_hbm, k0_v, k1_v, idx_v):
        wid = lax.axis_index("s") * num_cores + lax.axis_index("c")
        d0 = wid * per_w
        lane = lax.iota(jnp.int32, 16)

        for g in range(n_groups):
            dg = d0 + g * 16
            b_glob = (bs - k_rows) + dg // emb
            pltpu.sync_copy(k0_hbm.at[b_glob], k0_v)
            pltpu.sync_copy(k1_hbm.at[b_glob], k1_v)
            k0 = k0_v[...]
            k1 = k1_v[...]
            ks2 = k0 ^ k1 ^ jnp.int32(_C240)
            inj = (k1, ks2, k0, k1, ks2 + jnp.int32(-(2**31)))
            inj1 = (ks2 + jnp.int32(1), k0 + jnp.int32(2), k1 + jnp.int32(3),
                    ks2 + jnp.int32(4), k0 + jnp.int32(5))

            e_vec = lax.rem(dg, emb) + lane
            x1_base = e_vec * seq_len + k1

            def s_body(s, carry, x1_base=x1_base):
                run_m, run_s = carry
                x0 = k0
                x1 = x1_base + s
                for i in range(5):
                    for r in _ROT[i % 2]:
                        x0 = x0 + x1
                        x1 = (x1 << r) + lax.shift_right_logical(x1, 32 - r)
                        x1 = x1 ^ x0
                    x0 = x0 + inj[i]
                    x1 = x1 + inj1[i]
                ym = (x0 ^ x1) & jnp.int32(-512)
                upd = ym > run_m
                return (
                    jnp.maximum(ym, run_m),
                    jnp.where(upd, jnp.full((16,), s, dtype=jnp.int32), run_s),
                )

            init = (
                jnp.full((16,), -(2**31), dtype=jnp.int32),
                jnp.zeros((16,), dtype=jnp.int32),
            )
            run_m, run_s = lax.fori_loop(0, seq_len, s_body, init)
            idx_v[pl.ds(g * 16, 16)] = (b_glob * seq_len + run_s) * emb + e_vec

        pltpu.sync_copy(idx_v, out_hbm.at[pl.ds(wid * per_w, per_w)])

    return sc_sample


def kernel(output_tokens_from_bert):
    bs, seq_len, emb = output_tokens_from_bert.shape
    sample_key = jax.random.key(42)
    keys = jax.random.split(sample_key, bs)
    keys_i32 = lax.bitcast_convert_type(jax.random.key_data(keys), jnp.int32)

    info = plsc.get_sparse_core_info()
    n_workers = info.num_cores * info.num_subcores

    k_rows = SC_ROWS
    tc_rows = bs - k_rows

    head_idx = pl.pallas_call(
        functools.partial(_sample_kernel, seq_len=seq_len, emb=emb),
        grid=(tc_rows,),
        in_specs=[pl.BlockSpec(memory_space=pltpu.SMEM)],
        out_specs=pl.BlockSpec((1, 1, emb), lambda b: (b, 0, 0)),
        out_shape=jax.ShapeDtypeStruct((tc_rows, 1, emb), jnp.int32),
        compiler_params=pltpu.CompilerParams(
            dimension_semantics=("parallel",),
        ),
    )(keys_i32)

    k0_splat = jnp.broadcast_to(keys_i32[:, 0:1], (bs, 16))
    k1_splat = jnp.broadcast_to(keys_i32[:, 1:2], (bs, 16))
    tail_idx = _make_sc_sample(
        bs, seq_len, emb, k_rows, n_workers, info.num_cores
    )(k0_splat, k1_splat)

    flat_idx = jnp.concatenate(
        [head_idx.reshape(tc_rows * emb), tail_idx]
    )

    n_idx = bs * emb
    per_w = n_idx // n_workers
    tokens_flat = output_tokens_from_bert.reshape(bs * seq_len * emb)
    vals = _make_sc_gather(n_idx, per_w, n_workers, info.num_cores)(
        tokens_flat, flat_idx
    )
    return vals.reshape(bs, emb)


# speedup vs baseline: 1.0098x; 1.0098x over previous
import functools

import jax
import jax.numpy as jnp
from jax import lax
from jax.experimental import pallas as pl
from jax.experimental.pallas import tpu as pltpu
from jax.experimental.pallas import tpu_sc as plsc


E_BLK = 256
S_BLK = 64

_ROT = ((13, 15, 26, 6), (17, 29, 16, 24))
_C240 = 0x1BD11BDA


def _threefry2x32(k0, k1, x0, x1):
    ks = (k0, k1, k0 ^ k1 ^ jnp.int32(_C240))
    for i in range(5):
        for r in _ROT[i % 2]:
            x0 = x0 + x1
            x1 = (x1 << r) + lax.shift_right_logical(x1, 32 - r)
            x1 = x1 ^ x0
        inj0 = ks[(i + 1) % 3]
        if i == 4:
            inj0 = inj0 + jnp.int32(-(2**31))
        x0 = x0 + inj0
        x1 = x1 + ks[(i + 2) % 3] + jnp.int32(i + 1)
    return x0, x1


def _sample_kernel(keys_ref, idx_ref, *, seq_len, emb):
    b = pl.program_id(0)
    k0 = keys_ref[b, 0]
    k1 = keys_ref[b, 1]

    n_schunk = seq_len // S_BLK

    sub = lax.broadcasted_iota(jnp.int32, (S_BLK, E_BLK), 0)
    lane = lax.broadcasted_iota(jnp.int32, (S_BLK, E_BLK), 1)
    sb_bits = S_BLK.bit_length() - 1
    revsub = (S_BLK - 1) - sub

    x0_init = jnp.full((S_BLK, E_BLK), k0, dtype=jnp.int32)

    for ec in range(0, emb, E_BLK):
        f0k = (lane + ec) * seq_len + sub + k1

        def one_chunk(s0, carry, f0k=f0k):
            run_max, run_idx = carry
            o0, o1 = _threefry2x32(k0, k1, x0_init, f0k + s0)
            y = o0 ^ o1
            packed = (y & jnp.int32(-512)) + revsub
            c_pack = jnp.max(packed, axis=0, keepdims=True)
            c_max = c_pack & jnp.int32(-512)
            c_idx = (s0 + (S_BLK - 1)) - (c_pack & (S_BLK - 1))
            upd = c_max > run_max
            return (
                jnp.where(upd, c_max, run_max),
                jnp.where(upd, c_idx, run_idx),
            )

        def s_body(si, carry):
            s0 = si * (2 * S_BLK)
            return one_chunk(s0 + S_BLK, one_chunk(s0, carry))

        init = (
            jnp.full((1, E_BLK), -(2**31), dtype=jnp.int32),
            jnp.zeros((1, E_BLK), dtype=jnp.int32),
        )
        _, s_star = lax.fori_loop(0, n_schunk // 2, s_body, init)
        lane_row = lax.broadcasted_iota(jnp.int32, (1, E_BLK), 1) + ec
        idx_ref[0, :, pl.ds(ec, E_BLK)] = (
            (b * seq_len + s_star) * emb + lane_row
        )


def _make_sc_gather(n_idx, per_w, n_workers, num_cores):
    mesh = plsc.VectorSubcoreMesh(core_axis_name="c", subcore_axis_name="s")

    @functools.partial(
        pl.kernel,
        mesh=mesh,
        out_type=jax.ShapeDtypeStruct((n_idx,), jnp.float32),
        scratch_types=[
            pltpu.VMEM((per_w,), jnp.int32),
            pltpu.VMEM((per_w,), jnp.float32),
            pltpu.SemaphoreType.DMA,
        ],
    )
    def sc_gather(tokens_hbm, idx_hbm, out_hbm, idx_v, vals_v, sem):
        wid = lax.axis_index("s") * num_cores + lax.axis_index("c")
        base = wid * per_w
        pltpu.sync_copy(idx_hbm.at[pl.ds(base, per_w)], idx_v)
        pltpu.async_copy(tokens_hbm.at[idx_v], vals_v, sem).wait()
        pltpu.sync_copy(vals_v, out_hbm.at[pl.ds(base, per_w)])

    return sc_gather


SC_ROWS = 14


def _make_sc_sample(bs, seq_len, emb, k_rows, n_workers, num_cores):
    mesh = plsc.VectorSubcoreMesh(core_axis_name="c", subcore_axis_name="s")
    per_w = (k_rows * emb) // n_workers
    n_groups = per_w // 16

    @functools.partial(
        pl.kernel,
        mesh=mesh,
        out_type=jax.ShapeDtypeStruct((k_rows * emb,), jnp.int32),
        scratch_types=[
            pltpu.VMEM((16,), jnp.int32),
            pltpu.VMEM((16,), jnp.int32),
            pltpu.VMEM((16,), jnp.int32),
            pltpu.VMEM((16,), jnp.int32),
            pltpu.VMEM((per_w,), jnp.int32),
        ],
    )
    def sc_sample(k0_hbm, k1_hbm, out_hbm, k0a_v, k1a_v, k0b_v, k1b_v, idx_v):
        wid = lax.axis_index("s") * num_cores + lax.axis_index("c")
        d0 = wid * per_w
        lane = lax.iota(jnp.int32, 16)

        def group_setup(dg, k0_v, k1_v):
            b_glob = (bs - k_rows) + dg // emb
            pltpu.sync_copy(k0_hbm.at[b_glob], k0_v)
            pltpu.sync_copy(k1_hbm.at[b_glob], k1_v)
            k0 = k0_v[...]
            k1 = k1_v[...]
            ks2 = k0 ^ k1 ^ jnp.int32(_C240)
            inj = (k1, ks2, k0, k1, ks2 + jnp.int32(-(2**31)))
            inj1 = (ks2 + jnp.int32(1), k0 + jnp.int32(2), k1 + jnp.int32(3),
                    ks2 + jnp.int32(4), k0 + jnp.int32(5))
            e_vec = lax.rem(dg, emb) + lane
            x1_base = e_vec * seq_len + k1
            return k0, inj, inj1, e_vec, x1_base, b_glob

        def one_step(s, run_m, run_s, k0, inj, inj1, x1_base):
            x0 = k0
            x1 = x1_base + s
            for i in range(5):
                for r in _ROT[i % 2]:
                    x0 = x0 + x1
                    x1 = (x1 << r) + lax.shift_right_logical(x1, 32 - r)
                    x1 = x1 ^ x0
                x0 = x0 + inj[i]
                x1 = x1 + inj1[i]
            ym = (x0 ^ x1) & jnp.int32(-512)
            upd = ym > run_m
            return (
                jnp.maximum(ym, run_m),
                jnp.where(upd, jnp.full((16,), s, dtype=jnp.int32), run_s),
            )

        for g in range(0, n_groups, 2):
            dga = d0 + g * 16
            dgb = d0 + (g + 1) * 16
            k0a, inja, inj1a, e_a, x1ba, b_a = group_setup(dga, k0a_v, k1a_v)
            k0b, injb, inj1b, e_b, x1bb, b_b = group_setup(dgb, k0b_v, k1b_v)

            def s_body(s, carry):
                ma, sa, mb, sb = carry
                ma, sa = one_step(s, ma, sa, k0a, inja, inj1a, x1ba)
                mb, sb = one_step(s, mb, sb, k0b, injb, inj1b, x1bb)
                return (ma, sa, mb, sb)

            neg = jnp.full((16,), -(2**31), dtype=jnp.int32)
            zero = jnp.zeros((16,), dtype=jnp.int32)
            ma, sa, mb, sb = lax.fori_loop(
                0, seq_len, s_body, (neg, zero, neg, zero)
            )
            idx_v[pl.ds(g * 16, 16)] = (b_a * seq_len + sa) * emb + e_a
            idx_v[pl.ds((g + 1) * 16, 16)] = (b_b * seq_len + sb) * emb + e_b

        pltpu.sync_copy(idx_v, out_hbm.at[pl.ds(wid * per_w, per_w)])

    return sc_sample


def kernel(output_tokens_from_bert):
    bs, seq_len, emb = output_tokens_from_bert.shape
    sample_key = jax.random.key(42)
    keys = jax.random.split(sample_key, bs)
    keys_i32 = lax.bitcast_convert_type(jax.random.key_data(keys), jnp.int32)

    info = plsc.get_sparse_core_info()
    n_workers = info.num_cores * info.num_subcores

    k_rows = SC_ROWS
    tc_rows = bs - k_rows

    head_idx = pl.pallas_call(
        functools.partial(_sample_kernel, seq_len=seq_len, emb=emb),
        grid=(tc_rows,),
        in_specs=[pl.BlockSpec(memory_space=pltpu.SMEM)],
        out_specs=pl.BlockSpec((1, 1, emb), lambda b: (b, 0, 0)),
        out_shape=jax.ShapeDtypeStruct((tc_rows, 1, emb), jnp.int32),
        compiler_params=pltpu.CompilerParams(
            dimension_semantics=("parallel",),
        ),
    )(keys_i32)

    k0_splat = jnp.broadcast_to(keys_i32[:, 0:1], (bs, 16))
    k1_splat = jnp.broadcast_to(keys_i32[:, 1:2], (bs, 16))
    tail_idx = _make_sc_sample(
        bs, seq_len, emb, k_rows, n_workers, info.num_cores
    )(k0_splat, k1_splat)

    flat_idx = jnp.concatenate(
        [head_idx.reshape(tc_rows * emb), tail_idx]
    )

    n_idx = bs * emb
    per_w = n_idx // n_workers
    tokens_flat = output_tokens_from_bert.reshape(bs * seq_len * emb)
    vals = _make_sc_gather(n_idx, per_w, n_workers, info.num_cores)(
        tokens_flat, flat_idx
    )
    return vals.reshape(bs, emb)


# speedup vs baseline: 1.0305x; 1.0206x over previous
import functools

import jax
import jax.numpy as jnp
from jax import lax
from jax.experimental import pallas as pl
from jax.experimental.pallas import tpu as pltpu
from jax.experimental.pallas import tpu_sc as plsc


E_BLK = 256
S_BLK = 64

_ROT = ((13, 15, 26, 6), (17, 29, 16, 24))
_C240 = 0x1BD11BDA


def _threefry2x32(k0, k1, x0, x1):
    ks = (k0, k1, k0 ^ k1 ^ jnp.int32(_C240))
    for i in range(5):
        for r in _ROT[i % 2]:
            x0 = x0 + x1
            x1 = (x1 << r) + lax.shift_right_logical(x1, 32 - r)
            x1 = x1 ^ x0
        inj0 = ks[(i + 1) % 3]
        if i == 4:
            inj0 = inj0 + jnp.int32(-(2**31))
        x0 = x0 + inj0
        x1 = x1 + ks[(i + 2) % 3] + jnp.int32(i + 1)
    return x0, x1


def _sample_kernel(keys_ref, idx_ref, *, seq_len, emb):
    b = pl.program_id(0)
    k0 = keys_ref[b, 0]
    k1 = keys_ref[b, 1]

    n_schunk = seq_len // S_BLK

    sub = lax.broadcasted_iota(jnp.int32, (S_BLK, E_BLK), 0)
    lane = lax.broadcasted_iota(jnp.int32, (S_BLK, E_BLK), 1)
    sb_bits = S_BLK.bit_length() - 1
    revsub = (S_BLK - 1) - sub

    x0_init = jnp.full((S_BLK, E_BLK), k0, dtype=jnp.int32)

    for ec in range(0, emb, E_BLK):
        f0k = (lane + ec) * seq_len + sub + k1

        def one_chunk(s0, carry, f0k=f0k):
            run_max, run_idx = carry
            o0, o1 = _threefry2x32(k0, k1, x0_init, f0k + s0)
            y = o0 ^ o1
            packed = (y & jnp.int32(-512)) + revsub
            c_pack = jnp.max(packed, axis=0, keepdims=True)
            c_max = c_pack & jnp.int32(-512)
            c_idx = (s0 + (S_BLK - 1)) - (c_pack & (S_BLK - 1))
            upd = c_max > run_max
            return (
                jnp.where(upd, c_max, run_max),
                jnp.where(upd, c_idx, run_idx),
            )

        def s_body(si, carry):
            s0 = si * (2 * S_BLK)
            return one_chunk(s0 + S_BLK, one_chunk(s0, carry))

        init = (
            jnp.full((1, E_BLK), -(2**31), dtype=jnp.int32),
            jnp.zeros((1, E_BLK), dtype=jnp.int32),
        )
        _, s_star = lax.fori_loop(0, n_schunk // 2, s_body, init)
        lane_row = lax.broadcasted_iota(jnp.int32, (1, E_BLK), 1) + ec
        idx_ref[0, :, pl.ds(ec, E_BLK)] = (
            (b * seq_len + s_star) * emb + lane_row
        )


def _make_sc_gather(n_idx, per_w, n_workers, num_cores):
    mesh = plsc.VectorSubcoreMesh(core_axis_name="c", subcore_axis_name="s")

    @functools.partial(
        pl.kernel,
        mesh=mesh,
        out_type=jax.ShapeDtypeStruct((n_idx,), jnp.float32),
        scratch_types=[
            pltpu.VMEM((per_w,), jnp.int32),
            pltpu.VMEM((per_w,), jnp.float32),
            pltpu.SemaphoreType.DMA,
        ],
    )
    def sc_gather(tokens_hbm, idx_hbm, out_hbm, idx_v, vals_v, sem):
        wid = lax.axis_index("s") * num_cores + lax.axis_index("c")
        base = wid * per_w
        pltpu.sync_copy(idx_hbm.at[pl.ds(base, per_w)], idx_v)
        pltpu.async_copy(tokens_hbm.at[idx_v], vals_v, sem).wait()
        pltpu.sync_copy(vals_v, out_hbm.at[pl.ds(base, per_w)])

    return sc_gather


SC_ROWS = 14


def _make_sc_sample(bs, seq_len, emb, k_rows, n_workers, num_cores):
    mesh = plsc.VectorSubcoreMesh(core_axis_name="c", subcore_axis_name="s")
    per_w = (k_rows * emb) // n_workers
    n_groups = per_w // 16

    @functools.partial(
        pl.kernel,
        mesh=mesh,
        out_type=jax.ShapeDtypeStruct((k_rows * emb,), jnp.int32),
        scratch_types=[
            pltpu.VMEM((16,), jnp.int32),
            pltpu.VMEM((16,), jnp.int32),
            pltpu.VMEM((per_w,), jnp.int32),
        ],
    )
    def sc_sample(k0_hbm, k1_hbm, out_hbm, k0_v, k1_v, idx_v):
        wid = lax.axis_index("s") * num_cores + lax.axis_index("c")
        d0 = wid * per_w
        lane = lax.iota(jnp.int32, 16)

        for g in range(n_groups):
            dg = d0 + g * 16
            b_glob = (bs - k_rows) + dg // emb
            pltpu.sync_copy(k0_hbm.at[b_glob], k0_v)
            pltpu.sync_copy(k1_hbm.at[b_glob], k1_v)
            k0 = k0_v[...]
            k1 = k1_v[...]
            ks2 = k0 ^ k1 ^ jnp.int32(_C240)
            inj = (k1, ks2, k0, k1, ks2 + jnp.int32(-(2**31)))
            inj1 = (ks2 + jnp.int32(1), k0 + jnp.int32(2), k1 + jnp.int32(3),
                    ks2 + jnp.int32(4), k0 + jnp.int32(5))

            e_vec = lax.rem(dg, emb) + lane
            x1_base = e_vec * seq_len + k1

            def s_body(s, carry, x1_base=x1_base):
                run_m, run_s = carry
                x0 = k0
                x1 = x1_base + s
                for i in range(5):
                    for r in _ROT[i % 2]:
                        x0 = x0 + x1
                        x1 = (x1 << r) + lax.shift_right_logical(x1, 32 - r)
                        x1 = x1 ^ x0
                    x0 = x0 + inj[i]
                    x1 = x1 + inj1[i]
                ym = (x0 ^ x1) & jnp.int32(-512)
                upd = ym > run_m
                return (
                    jnp.maximum(ym, run_m),
                    jnp.where(upd, jnp.full((16,), s, dtype=jnp.int32), run_s),
                )

            init = (
                jnp.full((16,), -(2**31), dtype=jnp.int32),
                jnp.zeros((16,), dtype=jnp.int32),
            )
            run_m, run_s = lax.fori_loop(0, seq_len, s_body, init)
            idx_v[pl.ds(g * 16, 16)] = (b_glob * seq_len + run_s) * emb + e_vec

        pltpu.sync_copy(idx_v, out_hbm.at[pl.ds(wid * per_w, per_w)])

    return sc_sample


def kernel(output_tokens_from_bert):
    bs, seq_len, emb = output_tokens_from_bert.shape
    sample_key = jax.random.key(42)
    keys = jax.random.split(sample_key, bs)
    keys_i32 = lax.bitcast_convert_type(jax.random.key_data(keys), jnp.int32)

    info = plsc.get_sparse_core_info()
    n_workers = info.num_cores * info.num_subcores

    k_rows = SC_ROWS
    tc_rows = bs - k_rows

    head_idx = pl.pallas_call(
        functools.partial(_sample_kernel, seq_len=seq_len, emb=emb),
        grid=(tc_rows,),
        in_specs=[pl.BlockSpec(memory_space=pltpu.SMEM)],
        out_specs=pl.BlockSpec((1, 1, emb), lambda b: (b, 0, 0)),
        out_shape=jax.ShapeDtypeStruct((tc_rows, 1, emb), jnp.int32),
        compiler_params=pltpu.CompilerParams(
            dimension_semantics=("parallel",),
        ),
    )(keys_i32)

    k0_splat = jnp.broadcast_to(keys_i32[:, 0:1], (bs, 16))
    k1_splat = jnp.broadcast_to(keys_i32[:, 1:2], (bs, 16))
    tail_idx = _make_sc_sample(
        bs, seq_len, emb, k_rows, n_workers, info.num_cores
    )(k0_splat, k1_splat)

    flat_idx = jnp.concatenate(
        [head_idx.reshape(tc_rows * emb), tail_idx]
    )

    n_idx = bs * emb
    per_w = n_idx // n_workers
    tokens_flat = output_tokens_from_bert.reshape(bs * seq_len * emb)
    vals = _make_sc_gather(n_idx, per_w, n_workers, info.num_cores)(
        tokens_flat, flat_idx
    )
    return vals.reshape(bs, emb)
